# Initial kernel scaffold; baseline (speedup 1.0000x reference)
import jax
import jax.numpy as jnp
from jax.experimental import pallas as pl


def kernel(input_tokens, expert_assignments, weight):
    raise NotImplementedError("write your pallas kernel here")


# speedup vs baseline: 2.2432x; 2.2432x over previous
import functools

import jax
import jax.numpy as jnp
from jax import lax
from jax.experimental import pallas as pl
from jax.experimental.pallas import tpu as pltpu
from jax.experimental.pallas import tpu_sc as plsc

T = 4096
F = 1024
O = 1024
E = 8
BM = 512
NB = T // BM
BM_SHIFT = 9
NI = 16

AR = 32
AC = 128

_HI = lax.Precision.HIGHEST


def _meta_body(a_ref, pos_ref, meta_ref):
    a = a_ref[...]
    ic_k = lax.broadcasted_iota(jnp.int32, (AC, AC), 0)
    ic_j = lax.broadcasted_iota(jnp.int32, (AC, AC), 1)
    tri_c = (ic_k <= ic_j).astype(jnp.float32)
    ir_i = lax.broadcasted_iota(jnp.int32, (AR, AR), 0)
    ir_k = lax.broadcasted_iota(jnp.int32, (AR, AR), 1)
    tri_r = (ir_k < ir_i).astype(jnp.float32)

    pos = jnp.zeros((AR, AC), jnp.float32)
    running = jnp.zeros((1, 1), jnp.float32)
    ends = []
    for e in range(E):
        m = (a == e).astype(jnp.float32)
        cum_incl = lax.dot(m, tri_c, precision=_HI)
        row_tot = cum_incl[:, AC - 1:AC]
        row_pref = lax.dot(tri_r, row_tot, precision=_HI)
        excl = cum_incl - m + row_pref
        pos = pos + m * (running + excl)
        running = running + row_pref[AR - 1:AR, :] + row_tot[AR - 1:AR, :]
        ends.append(running)
    pos_ref[...] = pos.astype(jnp.int32)

    lane = lax.broadcasted_iota(jnp.int32, (1, NI), 1)
    cuts = jnp.where(lane < NB, lane.astype(jnp.float32) * BM, jnp.float32(T))
    for k in range(E - 1):
        cuts = jnp.where(lane == NB + k, ends[k], cuts)

    cB = jnp.broadcast_to(cuts, (NI, NI))
    iB = lax.broadcasted_iota(jnp.int32, (NI, NI), 0)
    jB = lax.broadcasted_iota(jnp.int32, (NI, NI), 1)
    eye = (iB == jB)
    c_col = jnp.sum(jnp.where(eye, cB, 0.0), axis=1, keepdims=True)
    cmp = (cB < c_col) | ((cB == c_col) & (jB < iB))
    rank_col = jnp.sum(cmp.astype(jnp.float32), axis=1, keepdims=True)
    onehot = rank_col == jB.astype(jnp.float32)
    sorted_row = jnp.sum(
        jnp.where(onehot, jnp.broadcast_to(c_col, (NI, NI)), 0.0),
        axis=0, keepdims=True)

    shift = (iB == jB + 1).astype(jnp.float32)
    hi = lax.dot(sorted_row, shift, precision=_HI)
    hi = jnp.where(lane == NI - 1, jnp.float32(T), hi)

    lo_i = sorted_row.astype(jnp.int32)
    hi_i = hi.astype(jnp.int32)
    block = jnp.clip(lax.shift_right_logical(lo_i, BM_SHIFT), 0, NB - 1)
    expert = jnp.zeros((1, NI), jnp.int32)
    for k in range(E - 1):
        expert = expert + (ends[k] <= sorted_row).astype(jnp.int32)
    expert = jnp.clip(expert, 0, E - 1)
    base = block * BM
    meta_ref[0:1, :] = block
    meta_ref[1:2, :] = expert
    meta_ref[2:3, :] = lo_i - base
    meta_ref[3:4, :] = hi_i - base


def _meta_call(a2d):
    return pl.pallas_call(
        _meta_body,
        out_shape=[
            jax.ShapeDtypeStruct((AR, AC), jnp.int32),
            jax.ShapeDtypeStruct((4, NI), jnp.int32),
        ],
    )(a2d)


def _mm_body(meta_ref, x_ref, w_ref, o_ref):
    i = pl.program_id(0)
    lo = meta_ref[2, i]
    hi = meta_ref[3, i]

    @pl.when(hi > lo)
    def _():
        y = lax.dot_general(
            x_ref[...], w_ref[0],
            (((1,), (1,)), ((), ())),
            preferred_element_type=jnp.float32)
        rows = lax.broadcasted_iota(jnp.int32, (BM, 1), 0)
        msk = (rows >= lo) & (rows < hi)
        o_ref[...] = jnp.where(msk, y, o_ref[...])


def _mm_call(meta, x_sorted, weight):
    grid_spec = pltpu.PrefetchScalarGridSpec(
        num_scalar_prefetch=1,
        grid=(NI,),
        in_specs=[
            pl.BlockSpec((BM, F), lambda i, meta: (meta[0, i], 0)),
            pl.BlockSpec((1, O, F), lambda i, meta: (meta[1, i], 0, 0)),
        ],
        out_specs=pl.BlockSpec((BM, O), lambda i, meta: (meta[0, i], 0)),
    )
    return pl.pallas_call(
        _mm_body,
        grid_spec=grid_spec,
        out_shape=jax.ShapeDtypeStruct((T, O), jnp.float32),
    )(meta, x_sorted, weight)


_NC = 2
_NS = 16
_NWK = _NC * _NS
_RPW = T // _NWK
_CHUNK = 64
_NCH = _RPW // _CHUNK

@functools.lru_cache(maxsize=1)
def _sc_kernels():
    mesh = plsc.VectorSubcoreMesh(
        core_axis_name="c", subcore_axis_name="s",
        num_cores=_NC, num_subcores=_NS)
    scratch = [
        pltpu.VMEM((_NCH, _CHUNK), jnp.int32),
        pltpu.VMEM((_CHUNK, F), jnp.float32),
        pltpu.SemaphoreType.DMA,
    ]

    @functools.partial(
        pl.kernel,
        out_type=jax.ShapeDtypeStruct((T, F), jnp.float32),
        mesh=mesh,
        scratch_types=scratch,
    )
    def scatter(x_hbm, pos_hbm, out_hbm, idx_v, rows_v, sem):
        wid = lax.axis_index("s") * _NC + lax.axis_index("c")
        pltpu.sync_copy(pos_hbm.at[pl.ds(wid * _NCH, _NCH)], idx_v)
        for j in range(_NCH):
            pltpu.sync_copy(
                x_hbm.at[pl.ds(wid * _RPW + j * _CHUNK, _CHUNK)], rows_v)
            pltpu.async_copy(rows_v, out_hbm.at[idx_v.at[j]], sem).wait()

    @functools.partial(
        pl.kernel,
        out_type=jax.ShapeDtypeStruct((T, O), jnp.float32),
        mesh=mesh,
        scratch_types=scratch,
    )
    def gather(y_hbm, pos_hbm, out_hbm, idx_v, rows_v, sem):
        wid = lax.axis_index("s") * _NC + lax.axis_index("c")
        pltpu.sync_copy(pos_hbm.at[pl.ds(wid * _NCH, _NCH)], idx_v)
        for j in range(_NCH):
            pltpu.async_copy(y_hbm.at[idx_v.at[j]], rows_v, sem).wait()
            pltpu.sync_copy(
                rows_v, out_hbm.at[pl.ds(wid * _RPW + j * _CHUNK, _CHUNK)])

    return scatter, gather


def kernel(input_tokens, expert_assignments, weight):
    scatter, gather = _sc_kernels()
    a2d = expert_assignments.astype(jnp.int32).reshape(AR, AC)
    pos2d, meta = _meta_call(a2d)
    pos = pos2d.reshape(T // _CHUNK, _CHUNK)
    x_sorted = scatter(input_tokens, pos)
    y_sorted = _mm_call(meta, x_sorted, weight)
    return gather(y_sorted, pos)


# speedup vs baseline: 2.2499x; 1.0030x over previous
import functools

import jax
import jax.numpy as jnp
from jax import lax
from jax.experimental import pallas as pl
from jax.experimental.pallas import tpu as pltpu
from jax.experimental.pallas import tpu_sc as plsc

T = 4096
F = 1024
O = 1024
E = 8
BM = 512
NB = T // BM
BM_SHIFT = 9
NI = 16

AR = 128
AC = 32

_HI = lax.Precision.HIGHEST


def _meta_body(a_ref, pos_ref, meta_ref):
    a = a_ref[...]
    ic_k = lax.broadcasted_iota(jnp.int32, (AC, AC), 0)
    ic_j = lax.broadcasted_iota(jnp.int32, (AC, AC), 1)
    tri_c = (ic_k <= ic_j).astype(jnp.float32)
    ir_i = lax.broadcasted_iota(jnp.int32, (AR, AR), 0)
    ir_k = lax.broadcasted_iota(jnp.int32, (AR, AR), 1)
    tri_r = (ir_k < ir_i).astype(jnp.float32)

    pos = jnp.zeros((AR, AC), jnp.float32)
    running = jnp.zeros((1, 1), jnp.float32)
    ends = []
    for e in range(E):
        m = (a == e).astype(jnp.float32)
        cum_incl = lax.dot(m, tri_c, precision=_HI)
        row_tot = cum_incl[:, AC - 1:AC]
        row_pref = lax.dot(tri_r, row_tot, precision=_HI)
        excl = cum_incl - m + row_pref
        pos = pos + m * (running + excl)
        running = running + row_pref[AR - 1:AR, :] + row_tot[AR - 1:AR, :]
        ends.append(running)
    pos_ref[...] = pos.astype(jnp.int32)

    lane = lax.broadcasted_iota(jnp.int32, (1, NI), 1)
    cuts = jnp.where(lane < NB, lane.astype(jnp.float32) * BM, jnp.float32(T))
    for k in range(E - 1):
        cuts = jnp.where(lane == NB + k, ends[k], cuts)

    cB = jnp.broadcast_to(cuts, (NI, NI))
    iB = lax.broadcasted_iota(jnp.int32, (NI, NI), 0)
    jB = lax.broadcasted_iota(jnp.int32, (NI, NI), 1)
    eye = (iB == jB)
    c_col = jnp.sum(jnp.where(eye, cB, 0.0), axis=1, keepdims=True)
    cmp = (cB < c_col) | ((cB == c_col) & (jB < iB))
    rank_col = jnp.sum(cmp.astype(jnp.float32), axis=1, keepdims=True)
    onehot = rank_col == jB.astype(jnp.float32)
    sorted_row = jnp.sum(
        jnp.where(onehot, jnp.broadcast_to(c_col, (NI, NI)), 0.0),
        axis=0, keepdims=True)

    shift = (iB == jB + 1).astype(jnp.float32)
    hi = lax.dot(sorted_row, shift, precision=_HI)
    hi = jnp.where(lane == NI - 1, jnp.float32(T), hi)

    lo_i = sorted_row.astype(jnp.int32)
    hi_i = hi.astype(jnp.int32)
    block = jnp.clip(lax.shift_right_logical(lo_i, BM_SHIFT), 0, NB - 1)
    expert = jnp.zeros((1, NI), jnp.int32)
    for k in range(E - 1):
        expert = expert + (ends[k] <= sorted_row).astype(jnp.int32)
    expert = jnp.clip(expert, 0, E - 1)
    base = block * BM
    meta_ref[0:1, :] = block
    meta_ref[1:2, :] = expert
    meta_ref[2:3, :] = lo_i - base
    meta_ref[3:4, :] = hi_i - base


def _meta_call(a2d):
    return pl.pallas_call(
        _meta_body,
        out_shape=[
            jax.ShapeDtypeStruct((AR, AC), jnp.int32),
            jax.ShapeDtypeStruct((4, NI), jnp.int32),
        ],
    )(a2d)


def _mm_body(meta_ref, x_ref, w_ref, o_ref):
    i = pl.program_id(0)
    lo = meta_ref[2, i]
    hi = meta_ref[3, i]

    @pl.when(hi > lo)
    def _():
        y = lax.dot_general(
            x_ref[...], w_ref[0],
            (((1,), (1,)), ((), ())),
            preferred_element_type=jnp.float32)
        rows = lax.broadcasted_iota(jnp.int32, (BM, 1), 0)
        msk = (rows >= lo) & (rows < hi)
        o_ref[...] = jnp.where(msk, y, o_ref[...])


def _mm_call(meta, x_sorted, weight):
    grid_spec = pltpu.PrefetchScalarGridSpec(
        num_scalar_prefetch=1,
        grid=(NI,),
        in_specs=[
            pl.BlockSpec((BM, F), lambda i, meta: (meta[0, i], 0)),
            pl.BlockSpec((1, O, F), lambda i, meta: (meta[1, i], 0, 0)),
        ],
        out_specs=pl.BlockSpec((BM, O), lambda i, meta: (meta[0, i], 0)),
    )
    return pl.pallas_call(
        _mm_body,
        grid_spec=grid_spec,
        out_shape=jax.ShapeDtypeStruct((T, O), jnp.float32),
    )(meta, x_sorted, weight)


_NC = 2
_NS = 16
_NWK = _NC * _NS
_RPW = T // _NWK
_CHUNK = 32
_NCH = _RPW // _CHUNK


@functools.lru_cache(maxsize=1)
def _sc_kernels():
    mesh = plsc.VectorSubcoreMesh(
        core_axis_name="c", subcore_axis_name="s",
        num_cores=_NC, num_subcores=_NS)
    scratch = [
        pltpu.VMEM((_NCH, _CHUNK), jnp.int32),
        pltpu.VMEM((_CHUNK, F), jnp.float32),
        pltpu.VMEM((_CHUNK, F), jnp.float32),
        pltpu.SemaphoreType.DMA,
        pltpu.SemaphoreType.DMA,
        pltpu.SemaphoreType.DMA,
        pltpu.SemaphoreType.DMA,
    ]

    def make(out_shape, indirect_in):
        @functools.partial(
            pl.kernel,
            out_type=jax.ShapeDtypeStruct(out_shape, jnp.float32),
            mesh=mesh,
            scratch_types=scratch,
        )
        def k(src_hbm, pos_hbm, out_hbm, idx_v, rows0, rows1,
              si0, si1, so0, so1):
            wid = lax.axis_index("s") * _NC + lax.axis_index("c")
            pltpu.sync_copy(pos_hbm.at[pl.ds(wid * _NCH, _NCH)], idx_v)
            bufs = (rows0, rows1)
            sin = (si0, si1)
            sout = (so0, so1)

            def lin(j):
                return pl.ds(wid * _RPW + j * _CHUNK, _CHUNK)

            def start_in(j):
                b = j % 2
                src = (src_hbm.at[idx_v.at[j]] if indirect_in
                       else src_hbm.at[lin(j)])
                return pltpu.async_copy(src, bufs[b], sin[b])

            def start_out(j):
                b = j % 2
                dst = (out_hbm.at[lin(j)] if indirect_in
                       else out_hbm.at[idx_v.at[j]])
                return pltpu.async_copy(bufs[b], dst, sout[b])

            cin = {0: start_in(0), 1: start_in(1)}
            cout = {}
            for j in range(_NCH):
                cin[j].wait()
                cout[j] = start_out(j)
                if j + 2 < _NCH:
                    cout[j].wait()
                    cin[j + 2] = start_in(j + 2)
            cout[_NCH - 2].wait()
            cout[_NCH - 1].wait()

        return k

    return make((T, F), False), make((T, O), True)


def kernel(input_tokens, expert_assignments, weight):
    scatter, gather = _sc_kernels()
    pos, meta = _meta_call(
        expert_assignments.astype(jnp.int32).reshape(AR, AC))
    x_sorted = scatter(input_tokens, pos)
    y_sorted = _mm_call(meta, x_sorted, weight)
    return gather(y_sorted, pos)
